# baseline (speedup 1.0000x reference)
import functools

import jax
import jax.numpy as jnp
from jax import lax
from jax.experimental import pallas as pl
from jax.experimental.pallas import tpu as pltpu
from jax.experimental.pallas import tpu_sc as plsc

N = 10000
NP = 10240
E = 320000
G = 256
NC = 2
NS = 16
B = 128
RPT_ALL = 80
EP = RPT_ALL * B * NC * NS
NPT = NP // NS

_SC_MESH = dict(core_axis_name="c", subcore_axis_name="s",
                num_cores=NC, num_subcores=NS)


def _deg_body(dst3d, ones_hbm, z128, out_hbm, acc, idx_v, ones_v):
    c = lax.axis_index("c")
    s = lax.axis_index("s")
    w = c * NS + s
    pltpu.sync_copy(ones_hbm, ones_v)
    pltpu.sync_copy(dst3d.at[w], idx_v)
    pltpu.sync_copy(z128.at[pl.ds(s * NPT, NPT)], acc.at[pl.ds(s * NPT, NPT)])
    plsc.subcore_barrier()

    def body(j, carry):
        pltpu.sync_copy(ones_v, acc.at[idx_v.at[j]], add=True)
        return carry

    lax.fori_loop(0, RPT_ALL, body, 0)
    plsc.subcore_barrier()
    pltpu.sync_copy(acc.at[pl.ds(s * NPT, NPT)],
                    out_hbm.at[c].at[pl.ds(s * NPT, NPT)])


def _sc_deg(dst3d, ones128, z128):
    return pl.kernel(
        _deg_body,
        out_type=jax.ShapeDtypeStruct((NC, NP, 128), jnp.float32),
        mesh=plsc.VectorSubcoreMesh(**_SC_MESH),
        scratch_types=[
            pltpu.VMEM_SHARED((NP, 128), jnp.float32),
            pltpu.VMEM((RPT_ALL, B), jnp.int32),
            pltpu.VMEM((B, 128), jnp.float32),
        ],
    )(dst3d, ones128, z128)


NBUF = 2
HALVES = 2
RPH = RPT_ALL // HALVES


def _scatter_body(nch, W, table, src3d, dst3d, zW, out_hbm,
                  acc, src_v, dst_v, *rest):
    bufs = rest[:NBUF]
    gsem = rest[NBUF:2 * NBUF]
    c = lax.axis_index("c")
    s = lax.axis_index("s")
    w = c * NS + s
    for ch in range(nch):
        pltpu.sync_copy(zW.at[pl.ds(s * NPT, NPT)], acc.at[pl.ds(s * NPT, NPT)])
        plsc.subcore_barrier()

        for half in range(HALVES):
            pltpu.sync_copy(src3d.at[w].at[pl.ds(half * RPH, RPH)], src_v)
            pltpu.sync_copy(dst3d.at[w].at[pl.ds(half * RPH, RPH)], dst_v)
            for b in range(NBUF):
                pltpu.async_copy(table.at[ch].at[src_v.at[b]], bufs[b],
                                 gsem[b])

            def body(k, carry):
                for b in range(NBUF):
                    j = k * NBUF + b
                    pltpu.make_async_copy(table.at[ch].at[src_v.at[j]],
                                          bufs[b], gsem[b]).wait()
                    pltpu.sync_copy(bufs[b], acc.at[dst_v.at[j]], add=True)
                    nj = j + NBUF

                    @pl.when(nj < RPH)
                    def _():
                        pltpu.async_copy(table.at[ch].at[src_v.at[nj]],
                                         bufs[b], gsem[b])
                return carry

            lax.fori_loop(0, RPH // NBUF, body, 0)
        plsc.subcore_barrier()
        pltpu.sync_copy(acc.at[pl.ds(s * NPT, NPT)],
                        out_hbm.at[c].at[ch].at[pl.ds(s * NPT, NPT)])
        plsc.subcore_barrier()


def _sc_scatter(table, src3d, dst3d, zW):
    nch, _, W = table.shape
    return pl.kernel(
        functools.partial(_scatter_body, nch, W),
        out_type=jax.ShapeDtypeStruct((NC, nch, NP, W), jnp.float32),
        mesh=plsc.VectorSubcoreMesh(**_SC_MESH),
        scratch_types=(
            [pltpu.VMEM_SHARED((NP, W), jnp.float32),
             pltpu.VMEM((RPH, B), jnp.int32),
             pltpu.VMEM((RPH, B), jnp.int32)]
            + [pltpu.VMEM((B, W), jnp.float32)] * NBUF
            + [pltpu.SemaphoreType.DMA] * NBUF
        ),
    )(table, src3d, dst3d, zW)


GPT = G // (NC * NS)
WIN = 128


def _pool_body(h_flat, batch_hbm, z_acc, out_flat, b_v, win_v, acc_v):
    c = lax.axis_index("c")
    s = lax.axis_index("s")
    w = c * NS + s
    g0 = w * GPT
    pltpu.sync_copy(batch_hbm, b_v)
    pltpu.sync_copy(z_acc, acc_v)

    def count_lt(g):
        def cbody(i, cnt):
            v = b_v[pl.ds(i * 16, 16)]
            return cnt + jnp.where(v < g, 1.0, 0.0)
        return jnp.sum(lax.fori_loop(0, N // 16, cbody,
                                     jnp.zeros((16,), jnp.float32))
                       ).astype(jnp.int32)

    bounds = [count_lt(g0 + t) for t in range(GPT + 1)]
    for gg in range(GPT):
        start, end = bounds[gg], bounds[gg + 1]
        nwin = (end - start + WIN - 1) // WIN

        def wbody(t, carry, start=start, end=end, gg=gg):
            ptr = start + t * WIN
            wstart = jnp.minimum(ptr, N - WIN)
            pltpu.sync_copy(h_flat.at[pl.ds(wstart * 512, WIN * 512)], win_v)
            lo = ptr - wstart
            nn = jnp.minimum(end - ptr, WIN)

            def nbody(i, carry2):
                r = lo + i
                for k in range(32):
                    a = acc_v[pl.ds(gg * 512 + k * 16, 16)]
                    v = win_v[pl.ds(r * 512 + k * 16, 16)]
                    acc_v[pl.ds(gg * 512 + k * 16, 16)] = jnp.maximum(a, v)
                return carry2

            lax.fori_loop(0, nn, nbody, 0)
            return carry

        lax.fori_loop(0, nwin, wbody, 0)
    pltpu.sync_copy(acc_v, out_flat.at[pl.ds(g0 * 512, GPT * 512)])


def _sc_pool(h_flat, batch, z_acc):
    return pl.kernel(
        _pool_body,
        out_type=jax.ShapeDtypeStruct((G * 512,), jnp.float32),
        compiler_params=pltpu.CompilerParams(needs_layout_passes=False),
        mesh=plsc.VectorSubcoreMesh(**_SC_MESH),
        scratch_types=[
            pltpu.VMEM((N,), jnp.int32),
            pltpu.VMEM((WIN * 512,), jnp.float32),
            pltpu.VMEM((GPT * 512,), jnp.float32),
        ],
    )(h_flat, batch, z_acc)


R = 1024


def _dinv_body(degparts_ref, o_ref):
    deg = degparts_ref[0, :, 0:1] + degparts_ref[1, :, 0:1] + 1.0
    o_ref[...] = lax.rsqrt(deg)


def _tc_dinv(degparts):
    return pl.pallas_call(
        _dinv_body,
        grid=(NP // R,),
        in_specs=[pl.BlockSpec((NC, R, 128), lambda i: (0, i, 0))],
        out_specs=pl.BlockSpec((R, 1), lambda i: (i, 0)),
        out_shape=jax.ShapeDtypeStruct((NP, 1), jnp.float32),
    )(degparts)


def _layer1_body(x_ref, w_ref, dinv_ref, o_ref):
    xw = jnp.dot(x_ref[...], w_ref[0], preferred_element_type=jnp.float32)
    o_ref[0] = xw * dinv_ref[...]


def _tc_layer1(x, w, dinv, nch_out):
    dout = w.shape[1]
    Wc = dout // nch_out
    wch = w.reshape(128, nch_out, Wc).transpose(1, 0, 2)
    return pl.pallas_call(
        _layer1_body,
        grid=(NP // R, nch_out),
        in_specs=[
            pl.BlockSpec((R, 128), lambda i, j: (i, 0)),
            pl.BlockSpec((1, 128, Wc), lambda i, j: (j, 0, 0)),
            pl.BlockSpec((R, 1), lambda i, j: (i, 0)),
        ],
        out_specs=pl.BlockSpec((1, R, Wc), lambda i, j: (j, i, 0)),
        out_shape=jax.ShapeDtypeStruct((nch_out, NP, Wc), jnp.float32),
    )(x, wch, dinv)


def _layer_body(nch_in, Wc_in, acc_ref, xwp_ref, dinv_ref, b_ref, w_ref, o_ref):
    dinv = dinv_ref[...]
    parts = []
    for cc in range(nch_in):
        asum = acc_ref[0, cc] + acc_ref[1, cc]
        hp = dinv * (asum + xwp_ref[cc]) + b_ref[0, cc * Wc_in:(cc + 1) * Wc_in]
        parts.append(jnp.maximum(hp, 0.0))
    h = jnp.concatenate(parts, axis=1) if nch_in > 1 else parts[0]
    xw = jnp.dot(h, w_ref[...], preferred_element_type=jnp.float32)
    o_ref[0] = xw * dinv


def _tc_layer(acc, xwp, dinv, b, w, nch_out):
    _, nch_in, _, Wc_in = acc.shape
    din, dout = w.shape
    Wc = dout // nch_out
    return pl.pallas_call(
        functools.partial(_layer_body, nch_in, Wc_in),
        grid=(NP // R, nch_out),
        in_specs=[
            pl.BlockSpec((NC, nch_in, R, Wc_in), lambda i, j: (0, 0, i, 0)),
            pl.BlockSpec((nch_in, R, Wc_in), lambda i, j: (0, i, 0)),
            pl.BlockSpec((R, 1), lambda i, j: (i, 0)),
            pl.BlockSpec((1, din), lambda i, j: (0, 0)),
            pl.BlockSpec((din, Wc), lambda i, j: (0, j)),
        ],
        out_specs=pl.BlockSpec((1, R, Wc), lambda i, j: (j, i, 0)),
        out_shape=jax.ShapeDtypeStruct((nch_out, NP, Wc), jnp.float32),
    )(acc, xwp, dinv, b, w)


def _final_body(nch_in, Wc_in, acc_ref, xwp_ref, dinv_ref, b_ref, o_ref):
    dinv = dinv_ref[...]
    parts = []
    for cc in range(nch_in):
        asum = acc_ref[0, cc] + acc_ref[1, cc]
        hp = dinv * (asum + xwp_ref[cc]) + b_ref[0, cc * Wc_in:(cc + 1) * Wc_in]
        parts.append(jnp.maximum(hp, 0.0))
    o_ref[...] = jnp.concatenate(parts, axis=1)


def _tc_final(acc, xwp, dinv, b):
    _, nch_in, _, Wc_in = acc.shape
    din = nch_in * Wc_in
    return pl.pallas_call(
        functools.partial(_final_body, nch_in, Wc_in),
        grid=(NP // R,),
        in_specs=[
            pl.BlockSpec((NC, nch_in, R, Wc_in), lambda i: (0, 0, i, 0)),
            pl.BlockSpec((nch_in, R, Wc_in), lambda i: (0, i, 0)),
            pl.BlockSpec((R, 1), lambda i: (i, 0)),
            pl.BlockSpec((1, din), lambda i: (0, 0)),
        ],
        out_specs=pl.BlockSpec((R, din), lambda i: (i, 0)),
        out_shape=jax.ShapeDtypeStruct((NP, din), jnp.float32),
    )(acc, xwp, dinv, b)


GB = 16


def _pool3(y, L):
    L3 = (L - 3) // 3 + 1
    yr = y[:, :L3 * 3].reshape(y.shape[0], L3, 3, y.shape[2])
    return jnp.maximum(jnp.maximum(yr[:, :, 0], yr[:, :, 1]), yr[:, :, 2])


def _cell_body(x_ref, k1_ref, b1_ref, k2_ref, b2_ref, k3_ref, b3_ref, o_ref):
    x = x_ref[...]
    y1 = jnp.zeros((GB, 728, 32), jnp.float32)
    for j in range(8):
        y1 = y1 + x[:, j:j + 728, :] * k1_ref[j]
    y1 = jnp.maximum(y1 + b1_ref[...], 0.0)
    m1 = _pool3(y1, 728)

    x2 = jnp.concatenate([m1[:, j:j + 235, :] for j in range(8)], axis=2)
    z2 = jnp.dot(x2.reshape(GB * 235, 256), k2_ref[...],
                 preferred_element_type=jnp.float32).reshape(GB, 235, 64)
    y2 = jnp.maximum(z2 + b2_ref[...], 0.0)
    m2 = _pool3(y2, 235)

    x3 = jnp.concatenate([m2[:, j:j + 71, :] for j in range(8)], axis=2)
    z3 = jnp.dot(x3.reshape(GB * 71, 512), k3_ref[...],
                 preferred_element_type=jnp.float32).reshape(GB, 71, 128)
    y3 = jnp.maximum(z3 + b3_ref[...], 0.0)
    o_ref[...] = _pool3(y3, 71)


def _tc_cell(xg, k1, b1, k2, b2, k3, b3):
    return pl.pallas_call(
        _cell_body,
        grid=(G // GB,),
        in_specs=[
            pl.BlockSpec((GB, 735, 1), lambda i: (i, 0, 0)),
            pl.BlockSpec((8, 1, 32), lambda i: (0, 0, 0)),
            pl.BlockSpec((1, 1, 32), lambda i: (0, 0, 0)),
            pl.BlockSpec((256, 64), lambda i: (0, 0)),
            pl.BlockSpec((1, 1, 64), lambda i: (0, 0, 0)),
            pl.BlockSpec((512, 128), lambda i: (0, 0)),
            pl.BlockSpec((1, 1, 128), lambda i: (0, 0, 0)),
        ],
        out_specs=pl.BlockSpec((GB, 23, 128), lambda i: (i, 0, 0)),
        out_shape=jax.ShapeDtypeStruct((G, 23, 128), jnp.float32),
    )(xg, k1, b1, k2, b2, k3, b3)


def _head_body(p_ref, cf_ref, lw1, lb1, lw2, lb2, cfw, cfb,
               fw1a, fw1b, fb1, fw2, fb2, fw3, fb3, o_ref):
    mm = lambda a, b: jnp.dot(a, b[...], preferred_element_type=jnp.float32)
    d = jnp.maximum(mm(p_ref[...], lw1) + lb1[...], 0.0)
    d = jnp.maximum(mm(d, lw2) + lb2[...], 0.0)
    cv = mm(cf_ref[...], cfw) + cfb[...]
    f = jnp.maximum(mm(d, fw1a) + mm(cv, fw1b) + fb1[...], 0.0)
    f = jnp.maximum(mm(f, fw2) + fb2[...], 0.0)
    z = mm(f, fw3) + fb3[...]
    o_ref[...] = jax.nn.sigmoid(z)


def _tc_head(pooled, cflat, lW1, lB1, lW2, lB2, cFW2, cFB,
             fW1a, fW1b, fB1, fW2, fB2, fW3, fB3):
    args = (pooled, cflat, lW1, lB1[None], lW2, lB2[None], cFW2, cFB[None],
            fW1a, fW1b, fB1[None], fW2, fB2[None], fW3, fB3[None])
    return pl.pallas_call(
        _head_body,
        out_shape=jax.ShapeDtypeStruct((G, 1), jnp.float32),
    )(*args)


def kernel(x, edge_index, batch, cell, gW1, gB1, gW2, gB2, gW3, gB3,
           lW1, lB1, lW2, lB2, cK1, cb1, cK2, cb2, cK3, cb3, cFW, cFB,
           fW1, fB1, fW2, fB2, fW3, fB3):
    ei = edge_index.astype(jnp.int32)
    ei = jnp.concatenate(
        [ei, jnp.full((2, EP - E), N, jnp.int32)], axis=1)
    src3d = ei[0].reshape(NC * NS, RPT_ALL, B)
    dst3d = ei[1].reshape(NC * NS, RPT_ALL, B)

    ones128 = jnp.ones((B, 128), jnp.float32)
    z128 = jnp.zeros((NP, 128), jnp.float32)
    zpool = jnp.zeros((GPT * 512,), jnp.float32)
    xp = jnp.zeros((NP, 128), jnp.float32).at[:N].set(x)

    degparts = _sc_deg(dst3d, ones128, z128)
    dinv = _tc_dinv(degparts)

    xwp1 = _tc_layer1(xp, gW1, dinv, 1)
    acc1 = _sc_scatter(xwp1, src3d, dst3d, z128)
    xwp2 = _tc_layer(acc1, xwp1, dinv, gB1[None], gW2, 2)
    acc2 = _sc_scatter(xwp2, src3d, dst3d, z128)
    xwp3 = _tc_layer(acc2, xwp2, dinv, gB2[None], gW3, 4)
    acc3 = _sc_scatter(xwp3, src3d, dst3d, z128)
    h3 = _tc_final(acc3, xwp3, dinv, gB3[None])

    pooled = _sc_pool(h3.reshape(-1), batch.astype(jnp.int32), zpool)
    pooled = pooled.reshape(G, 512)

    k1 = cK1.reshape(32, 8).T.reshape(8, 1, 32)
    k2 = cK2.transpose(2, 1, 0).reshape(256, 64)
    k3 = cK3.transpose(2, 1, 0).reshape(512, 128)
    cfeat = _tc_cell(cell.reshape(G, 735, 1), k1, cb1.reshape(1, 1, 32),
                     k2, cb2.reshape(1, 1, 64), k3, cb3.reshape(1, 1, 128))
    cflat = cfeat.reshape(G, 23 * 128)
    cFW2 = cFW.reshape(128, 23, 128).transpose(1, 0, 2).reshape(23 * 128, 128)

    return _tc_head(pooled, cflat, lW1, lB1, lW2, lB2, cFW2, cFB,
                    fW1[:128], fW1[128:], fB1, fW2, fB2, fW3, fB3)


# speedup vs baseline: 2.8282x; 2.8282x over previous
import functools

import jax
import jax.numpy as jnp
from jax import lax
from jax.experimental import pallas as pl
from jax.experimental.pallas import tpu as pltpu
from jax.experimental.pallas import tpu_sc as plsc

N = 10000
NP = 10240
E = 320000
G = 256
NC = 2
NS = 16
B = 128
RPT_ALL = 80
EP = RPT_ALL * B * NC * NS
NPT = NP // NS

_SC_MESH = dict(core_axis_name="c", subcore_axis_name="s",
                num_cores=NC, num_subcores=NS)


def _deg_body(dst3d, ones_hbm, z128, out_hbm, acc, idx_v, ones_v):
    c = lax.axis_index("c")
    s = lax.axis_index("s")
    w = c * NS + s
    pltpu.sync_copy(ones_hbm, ones_v)
    pltpu.sync_copy(dst3d.at[w], idx_v)
    pltpu.sync_copy(z128.at[pl.ds(s * NPT, NPT)], acc.at[pl.ds(s * NPT, NPT)])
    plsc.subcore_barrier()

    def body(j, carry):
        pltpu.sync_copy(ones_v, acc.at[idx_v.at[j]], add=True)
        return carry

    lax.fori_loop(0, RPT_ALL, body, 0)
    plsc.subcore_barrier()
    pltpu.sync_copy(acc.at[pl.ds(s * NPT, NPT)],
                    out_hbm.at[c].at[pl.ds(s * NPT, NPT)])


def _sc_deg(dst3d, ones128, z128):
    return pl.kernel(
        _deg_body,
        out_type=jax.ShapeDtypeStruct((NC, NP, 128), jnp.float32),
        mesh=plsc.VectorSubcoreMesh(**_SC_MESH),
        scratch_types=[
            pltpu.VMEM_SHARED((NP, 128), jnp.float32),
            pltpu.VMEM((RPT_ALL, B), jnp.int32),
            pltpu.VMEM((B, 128), jnp.float32),
        ],
    )(dst3d, ones128, z128)


NBUF = 2
HALVES = 2
RPH = RPT_ALL // HALVES


def _scatter_body(nch, W, table, src3d, dst3d, zW, out_hbm,
                  acc, src_v, dst_v, *rest):
    bufs = rest[:NBUF]
    gsem = rest[NBUF:2 * NBUF]
    c = lax.axis_index("c")
    s = lax.axis_index("s")
    w = c * NS + s
    for ch in range(nch):
        pltpu.sync_copy(zW.at[pl.ds(s * NPT, NPT)], acc.at[pl.ds(s * NPT, NPT)])
        plsc.subcore_barrier()

        for half in range(HALVES):
            pltpu.sync_copy(src3d.at[w].at[pl.ds(half * RPH, RPH)], src_v)
            pltpu.sync_copy(dst3d.at[w].at[pl.ds(half * RPH, RPH)], dst_v)
            for b in range(NBUF):
                pltpu.async_copy(table.at[ch].at[src_v.at[b]], bufs[b],
                                 gsem[b])

            def body(k, carry):
                for b in range(NBUF):
                    j = k * NBUF + b
                    pltpu.make_async_copy(table.at[ch].at[src_v.at[j]],
                                          bufs[b], gsem[b]).wait()
                    pltpu.sync_copy(bufs[b], acc.at[dst_v.at[j]], add=True)
                    nj = j + NBUF

                    @pl.when(nj < RPH)
                    def _():
                        pltpu.async_copy(table.at[ch].at[src_v.at[nj]],
                                         bufs[b], gsem[b])
                return carry

            lax.fori_loop(0, RPH // NBUF, body, 0)
        plsc.subcore_barrier()
        pltpu.sync_copy(acc.at[pl.ds(s * NPT, NPT)],
                        out_hbm.at[c].at[ch].at[pl.ds(s * NPT, NPT)])
        plsc.subcore_barrier()


def _sc_scatter(table, src3d, dst3d, zW):
    nch, _, W = table.shape
    return pl.kernel(
        functools.partial(_scatter_body, nch, W),
        out_type=jax.ShapeDtypeStruct((NC, nch, NP, W), jnp.float32),
        mesh=plsc.VectorSubcoreMesh(**_SC_MESH),
        scratch_types=(
            [pltpu.VMEM_SHARED((NP, W), jnp.float32),
             pltpu.VMEM((RPH, B), jnp.int32),
             pltpu.VMEM((RPH, B), jnp.int32)]
            + [pltpu.VMEM((B, W), jnp.float32)] * NBUF
            + [pltpu.SemaphoreType.DMA] * NBUF
        ),
    )(table, src3d, dst3d, zW)


GPT = G // (NC * NS)
WIN = 128


def _pool_body(h_flat, batch_hbm, z_acc, out_flat, b_v, win_v, acc_v):
    c = lax.axis_index("c")
    s = lax.axis_index("s")
    w = c * NS + s
    g0 = w * GPT
    pltpu.sync_copy(batch_hbm, b_v)
    pltpu.sync_copy(z_acc, acc_v)

    def count_lt(g):
        def cbody(i, cnt):
            v = b_v[pl.ds(i * 16, 16)]
            return cnt + jnp.where(v < g, 1.0, 0.0)
        return jnp.sum(lax.fori_loop(0, N // 16, cbody,
                                     jnp.zeros((16,), jnp.float32))
                       ).astype(jnp.int32)

    bounds = [count_lt(g0 + t) for t in range(GPT + 1)]
    for gg in range(GPT):
        start, end = bounds[gg], bounds[gg + 1]
        nwin = (end - start + WIN - 1) // WIN

        def wbody(t, carry, start=start, end=end, gg=gg):
            ptr = start + t * WIN
            wstart = jnp.minimum(ptr, N - WIN)
            pltpu.sync_copy(h_flat.at[pl.ds(wstart * 512, WIN * 512)], win_v)
            lo = ptr - wstart
            nn = jnp.minimum(end - ptr, WIN)

            def nbody(i, carry2):
                r = lo + i
                for k in range(32):
                    a = acc_v[pl.ds(gg * 512 + k * 16, 16)]
                    v = win_v[pl.ds(r * 512 + k * 16, 16)]
                    acc_v[pl.ds(gg * 512 + k * 16, 16)] = jnp.maximum(a, v)
                return carry2

            lax.fori_loop(0, nn, nbody, 0)
            return carry

        lax.fori_loop(0, nwin, wbody, 0)
    pltpu.sync_copy(acc_v, out_flat.at[pl.ds(g0 * 512, GPT * 512)])


def _sc_pool(h_flat, batch, z_acc):
    return pl.kernel(
        _pool_body,
        out_type=jax.ShapeDtypeStruct((G * 512,), jnp.float32),
        compiler_params=pltpu.CompilerParams(needs_layout_passes=False),
        mesh=plsc.VectorSubcoreMesh(**_SC_MESH),
        scratch_types=[
            pltpu.VMEM((N,), jnp.int32),
            pltpu.VMEM((WIN * 512,), jnp.float32),
            pltpu.VMEM((GPT * 512,), jnp.float32),
        ],
    )(h_flat, batch, z_acc)


R = 1024


def _dinv_body(degparts_ref, o_ref):
    deg = degparts_ref[0, :, 0:1] + degparts_ref[1, :, 0:1] + 1.0
    o_ref[...] = lax.rsqrt(deg)


def _tc_dinv(degparts):
    return pl.pallas_call(
        _dinv_body,
        grid=(NP // R,),
        in_specs=[pl.BlockSpec((NC, R, 128), lambda i: (0, i, 0))],
        out_specs=pl.BlockSpec((R, 1), lambda i: (i, 0)),
        out_shape=jax.ShapeDtypeStruct((NP, 1), jnp.float32),
    )(degparts)


def _layer1_body(x_ref, w_ref, dinv_ref, o_ref):
    xw = jnp.dot(x_ref[...], w_ref[0], preferred_element_type=jnp.float32)
    o_ref[0] = xw * dinv_ref[...]


def _tc_layer1(x, w, dinv, nch_out):
    dout = w.shape[1]
    Wc = dout // nch_out
    wch = w.reshape(128, nch_out, Wc).transpose(1, 0, 2)
    return pl.pallas_call(
        _layer1_body,
        grid=(NP // R, nch_out),
        in_specs=[
            pl.BlockSpec((R, 128), lambda i, j: (i, 0)),
            pl.BlockSpec((1, 128, Wc), lambda i, j: (j, 0, 0)),
            pl.BlockSpec((R, 1), lambda i, j: (i, 0)),
        ],
        out_specs=pl.BlockSpec((1, R, Wc), lambda i, j: (j, i, 0)),
        out_shape=jax.ShapeDtypeStruct((nch_out, NP, Wc), jnp.float32),
    )(x, wch, dinv)


def _layer_body(nch_in, Wc_in, acc_ref, xwp_ref, dinv_ref, b_ref, w_ref, o_ref):
    dinv = dinv_ref[...]
    parts = []
    for cc in range(nch_in):
        asum = acc_ref[0, cc] + acc_ref[1, cc]
        hp = dinv * (asum + xwp_ref[cc]) + b_ref[0, cc * Wc_in:(cc + 1) * Wc_in]
        parts.append(jnp.maximum(hp, 0.0))
    h = jnp.concatenate(parts, axis=1) if nch_in > 1 else parts[0]
    xw = jnp.dot(h, w_ref[...], preferred_element_type=jnp.float32)
    o_ref[0] = xw * dinv


def _tc_layer(acc, xwp, dinv, b, w, nch_out):
    _, nch_in, _, Wc_in = acc.shape
    din, dout = w.shape
    Wc = dout // nch_out
    return pl.pallas_call(
        functools.partial(_layer_body, nch_in, Wc_in),
        grid=(NP // R, nch_out),
        in_specs=[
            pl.BlockSpec((NC, nch_in, R, Wc_in), lambda i, j: (0, 0, i, 0)),
            pl.BlockSpec((nch_in, R, Wc_in), lambda i, j: (0, i, 0)),
            pl.BlockSpec((R, 1), lambda i, j: (i, 0)),
            pl.BlockSpec((1, din), lambda i, j: (0, 0)),
            pl.BlockSpec((din, Wc), lambda i, j: (0, j)),
        ],
        out_specs=pl.BlockSpec((1, R, Wc), lambda i, j: (j, i, 0)),
        out_shape=jax.ShapeDtypeStruct((nch_out, NP, Wc), jnp.float32),
    )(acc, xwp, dinv, b, w)


def _final_body(nch_in, Wc_in, acc_ref, xwp_ref, dinv_ref, b_ref, o_ref):
    dinv = dinv_ref[...]
    parts = []
    for cc in range(nch_in):
        asum = acc_ref[0, cc] + acc_ref[1, cc]
        hp = dinv * (asum + xwp_ref[cc]) + b_ref[0, cc * Wc_in:(cc + 1) * Wc_in]
        parts.append(jnp.maximum(hp, 0.0))
    o_ref[...] = jnp.concatenate(parts, axis=1)


def _tc_final(acc, xwp, dinv, b):
    _, nch_in, _, Wc_in = acc.shape
    din = nch_in * Wc_in
    return pl.pallas_call(
        functools.partial(_final_body, nch_in, Wc_in),
        grid=(NP // R,),
        in_specs=[
            pl.BlockSpec((NC, nch_in, R, Wc_in), lambda i: (0, 0, i, 0)),
            pl.BlockSpec((nch_in, R, Wc_in), lambda i: (0, i, 0)),
            pl.BlockSpec((R, 1), lambda i: (i, 0)),
            pl.BlockSpec((1, din), lambda i: (0, 0)),
        ],
        out_specs=pl.BlockSpec((R, din), lambda i: (i, 0)),
        out_shape=jax.ShapeDtypeStruct((NP, din), jnp.float32),
    )(acc, xwp, dinv, b)


GB = 16


def _pool3(y, L):
    L3 = (L - 3) // 3 + 1
    yr = y[:, :L3 * 3].reshape(y.shape[0], L3, 3, y.shape[2])
    return jnp.maximum(jnp.maximum(yr[:, :, 0], yr[:, :, 1]), yr[:, :, 2])


def _cell_body(x_ref, k1_ref, b1_ref, k2_ref, b2_ref, k3_ref, b3_ref, o_ref):
    x = x_ref[...]
    y1 = jnp.zeros((GB, 728, 32), jnp.float32)
    for j in range(8):
        y1 = y1 + x[:, j:j + 728, :] * k1_ref[j]
    y1 = jnp.maximum(y1 + b1_ref[...], 0.0)
    m1 = _pool3(y1, 728)

    x2 = jnp.concatenate([m1[:, j:j + 235, :] for j in range(8)], axis=2)
    z2 = jnp.dot(x2.reshape(GB * 235, 256), k2_ref[...],
                 preferred_element_type=jnp.float32).reshape(GB, 235, 64)
    y2 = jnp.maximum(z2 + b2_ref[...], 0.0)
    m2 = _pool3(y2, 235)

    x3 = jnp.concatenate([m2[:, j:j + 71, :] for j in range(8)], axis=2)
    z3 = jnp.dot(x3.reshape(GB * 71, 512), k3_ref[...],
                 preferred_element_type=jnp.float32).reshape(GB, 71, 128)
    y3 = jnp.maximum(z3 + b3_ref[...], 0.0)
    o_ref[...] = _pool3(y3, 71)


def _tc_cell(xg, k1, b1, k2, b2, k3, b3):
    return pl.pallas_call(
        _cell_body,
        grid=(G // GB,),
        in_specs=[
            pl.BlockSpec((GB, 735, 1), lambda i: (i, 0, 0)),
            pl.BlockSpec((8, 1, 32), lambda i: (0, 0, 0)),
            pl.BlockSpec((1, 1, 32), lambda i: (0, 0, 0)),
            pl.BlockSpec((256, 64), lambda i: (0, 0)),
            pl.BlockSpec((1, 1, 64), lambda i: (0, 0, 0)),
            pl.BlockSpec((512, 128), lambda i: (0, 0)),
            pl.BlockSpec((1, 1, 128), lambda i: (0, 0, 0)),
        ],
        out_specs=pl.BlockSpec((GB, 23, 128), lambda i: (i, 0, 0)),
        out_shape=jax.ShapeDtypeStruct((G, 23, 128), jnp.float32),
    )(xg, k1, b1, k2, b2, k3, b3)


def _head_body(p_ref, cf_ref, lw1, lb1, lw2, lb2, cfw, cfb,
               fw1a, fw1b, fb1, fw2, fb2, fw3, fb3, o_ref):
    mm = lambda a, b: jnp.dot(a, b[...], preferred_element_type=jnp.float32)
    d = jnp.maximum(mm(p_ref[...], lw1) + lb1[...], 0.0)
    d = jnp.maximum(mm(d, lw2) + lb2[...], 0.0)
    cv = mm(cf_ref[...], cfw) + cfb[...]
    f = jnp.maximum(mm(d, fw1a) + mm(cv, fw1b) + fb1[...], 0.0)
    f = jnp.maximum(mm(f, fw2) + fb2[...], 0.0)
    z = mm(f, fw3) + fb3[...]
    o_ref[...] = jax.nn.sigmoid(z)


def _tc_head(pooled, cflat, lW1, lB1, lW2, lB2, cFW2, cFB,
             fW1a, fW1b, fB1, fW2, fB2, fW3, fB3):
    args = (pooled, cflat, lW1, lB1[None], lW2, lB2[None], cFW2, cFB[None],
            fW1a, fW1b, fB1[None], fW2, fB2[None], fW3, fB3[None])
    return pl.pallas_call(
        _head_body,
        out_shape=jax.ShapeDtypeStruct((G, 1), jnp.float32),
    )(*args)


def kernel(x, edge_index, batch, cell, gW1, gB1, gW2, gB2, gW3, gB3,
           lW1, lB1, lW2, lB2, cK1, cb1, cK2, cb2, cK3, cb3, cFW, cFB,
           fW1, fB1, fW2, fB2, fW3, fB3):
    ei = edge_index.astype(jnp.int32)
    pad = N + jnp.arange(EP - E, dtype=jnp.int32) % (NP - N)
    ei = jnp.concatenate([ei, jnp.stack([pad, pad])], axis=1)
    src3d = ei[0].reshape(NC * NS, RPT_ALL, B)
    dst3d = ei[1].reshape(NC * NS, RPT_ALL, B)

    ones128 = jnp.ones((B, 128), jnp.float32)
    z128 = jnp.zeros((NP, 128), jnp.float32)
    zpool = jnp.zeros((GPT * 512,), jnp.float32)
    xp = jnp.zeros((NP, 128), jnp.float32).at[:N].set(x)

    degparts = _sc_deg(dst3d, ones128, z128)
    dinv = _tc_dinv(degparts)

    xwp1 = _tc_layer1(xp, gW1, dinv, 1)
    acc1 = _sc_scatter(xwp1, src3d, dst3d, z128)
    xwp2 = _tc_layer(acc1, xwp1, dinv, gB1[None], gW2, 2)
    acc2 = _sc_scatter(xwp2, src3d, dst3d, z128)
    xwp3 = _tc_layer(acc2, xwp2, dinv, gB2[None], gW3, 4)
    acc3 = _sc_scatter(xwp3, src3d, dst3d, z128)
    h3 = _tc_final(acc3, xwp3, dinv, gB3[None])

    pooled = _sc_pool(h3.reshape(-1), batch.astype(jnp.int32), zpool)
    pooled = pooled.reshape(G, 512)

    k1 = cK1.reshape(32, 8).T.reshape(8, 1, 32)
    k2 = cK2.transpose(2, 1, 0).reshape(256, 64)
    k3 = cK3.transpose(2, 1, 0).reshape(512, 128)
    cfeat = _tc_cell(cell.reshape(G, 735, 1), k1, cb1.reshape(1, 1, 32),
                     k2, cb2.reshape(1, 1, 64), k3, cb3.reshape(1, 1, 128))
    cflat = cfeat.reshape(G, 23 * 128)
    cFW2 = cFW.reshape(128, 23, 128).transpose(1, 0, 2).reshape(23 * 128, 128)

    return _tc_head(pooled, cflat, lW1, lB1, lW2, lB2, cFW2, cFB,
                    fW1[:128], fW1[128:], fB1, fW2, fB2, fW3, fB3)


# speedup vs baseline: 2.8287x; 1.0002x over previous
import functools

import jax
import jax.numpy as jnp
from jax import lax
from jax.experimental import pallas as pl
from jax.experimental.pallas import tpu as pltpu
from jax.experimental.pallas import tpu_sc as plsc

N = 10000
NP = 10240
E = 320000
G = 256
NC = 2
NS = 16
B = 128
RPT_ALL = 80
EP = RPT_ALL * B * NC * NS
NPT = NP // NS

_SC_MESH = dict(core_axis_name="c", subcore_axis_name="s",
                num_cores=NC, num_subcores=NS)


def _deg_body(dst3d, ones_hbm, z128, out_hbm, acc, idx_v, ones_v):
    c = lax.axis_index("c")
    s = lax.axis_index("s")
    w = c * NS + s
    pltpu.sync_copy(ones_hbm, ones_v)
    pltpu.sync_copy(dst3d.at[w], idx_v)
    pltpu.sync_copy(z128.at[pl.ds(s * NPT, NPT)], acc.at[pl.ds(s * NPT, NPT)])
    plsc.subcore_barrier()

    def body(j, carry):
        pltpu.sync_copy(ones_v, acc.at[idx_v.at[j]], add=True)
        return carry

    lax.fori_loop(0, RPT_ALL, body, 0)
    plsc.subcore_barrier()
    pltpu.sync_copy(acc.at[pl.ds(s * NPT, NPT)],
                    out_hbm.at[c].at[pl.ds(s * NPT, NPT)])


def _sc_deg(dst3d, ones128, z128):
    return pl.kernel(
        _deg_body,
        out_type=jax.ShapeDtypeStruct((NC, NP, 128), jnp.float32),
        mesh=plsc.VectorSubcoreMesh(**_SC_MESH),
        scratch_types=[
            pltpu.VMEM_SHARED((NP, 128), jnp.float32),
            pltpu.VMEM((RPT_ALL, B), jnp.int32),
            pltpu.VMEM((B, 128), jnp.float32),
        ],
    )(dst3d, ones128, z128)


NBUF = 2
HALVES = 2
RPH = RPT_ALL // HALVES


def _scatter_body(nch, W, table, src3d, dst3d, zW, out_hbm,
                  acc, src_v, dst_v, *rest):
    bufs = rest[:NBUF]
    gsem = rest[NBUF:2 * NBUF]
    c = lax.axis_index("c")
    s = lax.axis_index("s")
    w = c * NS + s
    for ch in range(nch):
        pltpu.sync_copy(zW.at[pl.ds(s * NPT, NPT)], acc.at[pl.ds(s * NPT, NPT)])
        plsc.subcore_barrier()

        for half in range(HALVES):
            pltpu.sync_copy(src3d.at[w].at[pl.ds(half * RPH, RPH)], src_v)
            pltpu.sync_copy(dst3d.at[w].at[pl.ds(half * RPH, RPH)], dst_v)
            for b in range(NBUF):
                pltpu.async_copy(table.at[ch].at[src_v.at[b]], bufs[b],
                                 gsem[b])

            def body(k, carry):
                for b in range(NBUF):
                    j = k * NBUF + b
                    pltpu.make_async_copy(table.at[ch].at[src_v.at[j]],
                                          bufs[b], gsem[b]).wait()
                    pltpu.sync_copy(bufs[b], acc.at[dst_v.at[j]], add=True)
                    nj = j + NBUF

                    @pl.when(nj < RPH)
                    def _():
                        pltpu.async_copy(table.at[ch].at[src_v.at[nj]],
                                         bufs[b], gsem[b])
                return carry

            lax.fori_loop(0, RPH // NBUF, body, 0)
        plsc.subcore_barrier()
        pltpu.sync_copy(acc.at[pl.ds(s * NPT, NPT)],
                        out_hbm.at[c].at[ch].at[pl.ds(s * NPT, NPT)])
        plsc.subcore_barrier()


def _sc_scatter(table, src3d, dst3d, zW):
    nch, _, W = table.shape
    return pl.kernel(
        functools.partial(_scatter_body, nch, W),
        out_type=jax.ShapeDtypeStruct((NC, nch, NP, W), jnp.float32),
        mesh=plsc.VectorSubcoreMesh(**_SC_MESH),
        scratch_types=(
            [pltpu.VMEM_SHARED((NP, W), jnp.float32),
             pltpu.VMEM((RPH, B), jnp.int32),
             pltpu.VMEM((RPH, B), jnp.int32)]
            + [pltpu.VMEM((B, W), jnp.float32)] * NBUF
            + [pltpu.SemaphoreType.DMA] * NBUF
        ),
    )(table, src3d, dst3d, zW)


GPT = G // (NC * NS)
WIN = 128


def _pool_body(h_flat, batch_hbm, z_acc, out_flat, b_v, win_v, acc_v):
    c = lax.axis_index("c")
    s = lax.axis_index("s")
    w = c * NS + s
    g0 = w * GPT
    pltpu.sync_copy(batch_hbm, b_v)
    pltpu.sync_copy(z_acc, acc_v)

    def count_lt(g):
        def cbody(i, cnt):
            v = b_v[pl.ds(i * 16, 16)]
            return cnt + jnp.where(v < g, 1.0, 0.0)
        return jnp.sum(lax.fori_loop(0, N // 16, cbody,
                                     jnp.zeros((16,), jnp.float32))
                       ).astype(jnp.int32)

    bounds = [count_lt(g0 + t) for t in range(GPT + 1)]
    for gg in range(GPT):
        start, end = bounds[gg], bounds[gg + 1]
        nwin = (end - start + WIN - 1) // WIN

        def wbody(t, carry, start=start, end=end, gg=gg):
            ptr = start + t * WIN
            wstart = jnp.minimum(ptr, N - WIN)
            pltpu.sync_copy(h_flat.at[pl.ds(wstart * 512, WIN * 512)], win_v)
            lo = ptr - wstart
            nn = jnp.minimum(end - ptr, WIN)

            def nbody(i, carry2):
                r = lo + i
                for k in range(32):
                    a = acc_v[pl.ds(gg * 512 + k * 16, 16)]
                    v = win_v[pl.ds(r * 512 + k * 16, 16)]
                    acc_v[pl.ds(gg * 512 + k * 16, 16)] = jnp.maximum(a, v)
                return carry2

            lax.fori_loop(0, nn, nbody, 0)
            return carry

        lax.fori_loop(0, nwin, wbody, 0)
    pltpu.sync_copy(acc_v, out_flat.at[pl.ds(g0 * 512, GPT * 512)])


def _sc_pool(h_flat, batch, z_acc):
    return pl.kernel(
        _pool_body,
        out_type=jax.ShapeDtypeStruct((G * 512,), jnp.float32),
        compiler_params=pltpu.CompilerParams(needs_layout_passes=False),
        mesh=plsc.VectorSubcoreMesh(**_SC_MESH),
        scratch_types=[
            pltpu.VMEM((N,), jnp.int32),
            pltpu.VMEM((WIN * 512,), jnp.float32),
            pltpu.VMEM((GPT * 512,), jnp.float32),
        ],
    )(h_flat, batch, z_acc)


R = 1024


def _dinv_body(degparts_ref, o_ref):
    deg = degparts_ref[0, :, 0:1] + degparts_ref[1, :, 0:1] + 1.0
    o_ref[...] = lax.rsqrt(deg)


def _tc_dinv(degparts):
    return pl.pallas_call(
        _dinv_body,
        grid=(NP // R,),
        in_specs=[pl.BlockSpec((NC, R, 128), lambda i: (0, i, 0))],
        out_specs=pl.BlockSpec((R, 1), lambda i: (i, 0)),
        out_shape=jax.ShapeDtypeStruct((NP, 1), jnp.float32),
    )(degparts)


def _layer1_body(x_ref, w_ref, dinv_ref, o_ref):
    xw = jnp.dot(x_ref[...], w_ref[0], preferred_element_type=jnp.float32)
    o_ref[0] = xw * dinv_ref[...]


def _tc_layer1(x, w, dinv, nch_out):
    dout = w.shape[1]
    Wc = dout // nch_out
    wch = w.reshape(128, nch_out, Wc).transpose(1, 0, 2)
    return pl.pallas_call(
        _layer1_body,
        grid=(NP // R, nch_out),
        in_specs=[
            pl.BlockSpec((R, 128), lambda i, j: (i, 0)),
            pl.BlockSpec((1, 128, Wc), lambda i, j: (j, 0, 0)),
            pl.BlockSpec((R, 1), lambda i, j: (i, 0)),
        ],
        out_specs=pl.BlockSpec((1, R, Wc), lambda i, j: (j, i, 0)),
        out_shape=jax.ShapeDtypeStruct((nch_out, NP, Wc), jnp.float32),
    )(x, wch, dinv)


def _layer_body(nch_in, Wc_in, acc_ref, xwp_ref, dinv_ref, b_ref, w_ref, o_ref):
    dinv = dinv_ref[...]
    parts = []
    for cc in range(nch_in):
        asum = acc_ref[0, cc] + acc_ref[1, cc]
        hp = dinv * (asum + xwp_ref[cc]) + b_ref[0, cc * Wc_in:(cc + 1) * Wc_in]
        parts.append(jnp.maximum(hp, 0.0))
    h = jnp.concatenate(parts, axis=1) if nch_in > 1 else parts[0]
    xw = jnp.dot(h, w_ref[...], preferred_element_type=jnp.float32)
    o_ref[0] = xw * dinv


def _tc_layer(acc, xwp, dinv, b, w, nch_out):
    _, nch_in, _, Wc_in = acc.shape
    din, dout = w.shape
    Wc = dout // nch_out
    return pl.pallas_call(
        functools.partial(_layer_body, nch_in, Wc_in),
        grid=(NP // R, nch_out),
        in_specs=[
            pl.BlockSpec((NC, nch_in, R, Wc_in), lambda i, j: (0, 0, i, 0)),
            pl.BlockSpec((nch_in, R, Wc_in), lambda i, j: (0, i, 0)),
            pl.BlockSpec((R, 1), lambda i, j: (i, 0)),
            pl.BlockSpec((1, din), lambda i, j: (0, 0)),
            pl.BlockSpec((din, Wc), lambda i, j: (0, j)),
        ],
        out_specs=pl.BlockSpec((1, R, Wc), lambda i, j: (j, i, 0)),
        out_shape=jax.ShapeDtypeStruct((nch_out, NP, Wc), jnp.float32),
    )(acc, xwp, dinv, b, w)


def _final_body(nch_in, Wc_in, acc_ref, xwp_ref, dinv_ref, b_ref, o_ref):
    dinv = dinv_ref[...]
    parts = []
    for cc in range(nch_in):
        asum = acc_ref[0, cc] + acc_ref[1, cc]
        hp = dinv * (asum + xwp_ref[cc]) + b_ref[0, cc * Wc_in:(cc + 1) * Wc_in]
        parts.append(jnp.maximum(hp, 0.0))
    o_ref[...] = jnp.concatenate(parts, axis=1)


def _tc_final(acc, xwp, dinv, b):
    _, nch_in, _, Wc_in = acc.shape
    din = nch_in * Wc_in
    return pl.pallas_call(
        functools.partial(_final_body, nch_in, Wc_in),
        grid=(NP // R,),
        in_specs=[
            pl.BlockSpec((NC, nch_in, R, Wc_in), lambda i: (0, 0, i, 0)),
            pl.BlockSpec((nch_in, R, Wc_in), lambda i: (0, i, 0)),
            pl.BlockSpec((R, 1), lambda i: (i, 0)),
            pl.BlockSpec((1, din), lambda i: (0, 0)),
        ],
        out_specs=pl.BlockSpec((R, din), lambda i: (i, 0)),
        out_shape=jax.ShapeDtypeStruct((NP, din), jnp.float32),
    )(acc, xwp, dinv, b)


GB = 16


def _pool3(y, L):
    L3 = (L - 3) // 3 + 1
    yr = y[:, :L3 * 3].reshape(y.shape[0], L3, 3, y.shape[2])
    return jnp.maximum(jnp.maximum(yr[:, :, 0], yr[:, :, 1]), yr[:, :, 2])


def _cell_body(x_ref, k1_ref, b1_ref, k2_ref, b2_ref, k3_ref, b3_ref, o_ref):
    x = x_ref[...]
    y1 = jnp.zeros((GB, 728, 32), jnp.float32)
    for j in range(8):
        y1 = y1 + x[:, j:j + 728, :] * k1_ref[j]
    y1 = jnp.maximum(y1 + b1_ref[...], 0.0)
    m1 = _pool3(y1, 728)

    x2 = jnp.concatenate([m1[:, j:j + 235, :] for j in range(8)], axis=2)
    z2 = jnp.dot(x2.reshape(GB * 235, 256), k2_ref[...],
                 preferred_element_type=jnp.float32).reshape(GB, 235, 64)
    y2 = jnp.maximum(z2 + b2_ref[...], 0.0)
    m2 = _pool3(y2, 235)

    x3 = jnp.concatenate([m2[:, j:j + 71, :] for j in range(8)], axis=2)
    z3 = jnp.dot(x3.reshape(GB * 71, 512), k3_ref[...],
                 preferred_element_type=jnp.float32).reshape(GB, 71, 128)
    y3 = jnp.maximum(z3 + b3_ref[...], 0.0)
    o_ref[...] = _pool3(y3, 71)


def _tc_cell(xg, k1, b1, k2, b2, k3, b3):
    return pl.pallas_call(
        _cell_body,
        grid=(G // GB,),
        in_specs=[
            pl.BlockSpec((GB, 735, 1), lambda i: (i, 0, 0)),
            pl.BlockSpec((8, 1, 32), lambda i: (0, 0, 0)),
            pl.BlockSpec((1, 1, 32), lambda i: (0, 0, 0)),
            pl.BlockSpec((256, 64), lambda i: (0, 0)),
            pl.BlockSpec((1, 1, 64), lambda i: (0, 0, 0)),
            pl.BlockSpec((512, 128), lambda i: (0, 0)),
            pl.BlockSpec((1, 1, 128), lambda i: (0, 0, 0)),
        ],
        out_specs=pl.BlockSpec((GB, 23, 128), lambda i: (i, 0, 0)),
        out_shape=jax.ShapeDtypeStruct((G, 23, 128), jnp.float32),
    )(xg, k1, b1, k2, b2, k3, b3)


def _head_body(p_ref, cf_ref, lw1, lb1, lw2, lb2, cfw, cfb,
               fw1a, fw1b, fb1, fw2, fb2, fw3, fb3, o_ref):
    mm = lambda a, b: jnp.dot(a, b[...], preferred_element_type=jnp.float32)
    d = jnp.maximum(mm(p_ref[...], lw1) + lb1[...], 0.0)
    d = jnp.maximum(mm(d, lw2) + lb2[...], 0.0)
    cv = mm(cf_ref[...], cfw) + cfb[...]
    f = jnp.maximum(mm(d, fw1a) + mm(cv, fw1b) + fb1[...], 0.0)
    f = jnp.maximum(mm(f, fw2) + fb2[...], 0.0)
    z = mm(f, fw3) + fb3[...]
    o_ref[...] = jax.nn.sigmoid(z)


def _tc_head(pooled, cflat, lW1, lB1, lW2, lB2, cFW2, cFB,
             fW1a, fW1b, fB1, fW2, fB2, fW3, fB3):
    args = (pooled, cflat, lW1, lB1[None], lW2, lB2[None], cFW2, cFB[None],
            fW1a, fW1b, fB1[None], fW2, fB2[None], fW3, fB3[None])
    return pl.pallas_call(
        _head_body,
        out_shape=jax.ShapeDtypeStruct((G, 1), jnp.float32),
    )(*args)


def kernel(x, edge_index, batch, cell, gW1, gB1, gW2, gB2, gW3, gB3,
           lW1, lB1, lW2, lB2, cK1, cb1, cK2, cb2, cK3, cb3, cFW, cFB,
           fW1, fB1, fW2, fB2, fW3, fB3):
    ei = edge_index.astype(jnp.int32)
    pad = N + jnp.arange(EP - E, dtype=jnp.int32) % (NP - N)
    ei = jnp.concatenate([ei, jnp.stack([pad, pad])], axis=1)
    src3d = ei[0].reshape(NC * NS, RPT_ALL, B)
    dst3d = ei[1].reshape(NC * NS, RPT_ALL, B)

    ones128 = jnp.ones((B, 128), jnp.float32)
    z128 = jnp.zeros((NP, 128), jnp.float32)
    zpool = jnp.zeros((GPT * 512,), jnp.float32)
    xp = jnp.zeros((NP, 128), jnp.float32).at[:N].set(x)

    k1 = cK1.reshape(32, 8).T.reshape(8, 1, 32)
    k2 = cK2.transpose(2, 1, 0).reshape(256, 64)
    k3 = cK3.transpose(2, 1, 0).reshape(512, 128)
    cfeat = _tc_cell(cell.reshape(G, 735, 1), k1, cb1.reshape(1, 1, 32),
                     k2, cb2.reshape(1, 1, 64), k3, cb3.reshape(1, 1, 128))
    cflat = cfeat.reshape(G, 23 * 128)
    cFW2 = cFW.reshape(128, 23, 128).transpose(1, 0, 2).reshape(23 * 128, 128)

    degparts = _sc_deg(dst3d, ones128, z128)
    dinv = _tc_dinv(degparts)

    xwp1 = _tc_layer1(xp, gW1, dinv, 1)
    acc1 = _sc_scatter(xwp1, src3d, dst3d, z128)
    xwp2 = _tc_layer(acc1, xwp1, dinv, gB1[None], gW2, 2)
    acc2 = _sc_scatter(xwp2, src3d, dst3d, z128)
    xwp3 = _tc_layer(acc2, xwp2, dinv, gB2[None], gW3, 4)
    acc3 = _sc_scatter(xwp3, src3d, dst3d, z128)
    h3 = _tc_final(acc3, xwp3, dinv, gB3[None])

    pooled = _sc_pool(h3.reshape(-1), batch.astype(jnp.int32), zpool)
    pooled = pooled.reshape(G, 512)

    return _tc_head(pooled, cflat, lW1, lB1, lW2, lB2, cFW2, cFB,
                    fW1[:128], fW1[128:], fB1, fW2, fB2, fW3, fB3)


# speedup vs baseline: 3.4923x; 1.2346x over previous
import functools

import jax
import jax.numpy as jnp
from jax import lax
from jax.experimental import pallas as pl
from jax.experimental.pallas import tpu as pltpu
from jax.experimental.pallas import tpu_sc as plsc

N = 10000
NP = 10240
E = 320000
G = 256
NC = 2
NS = 16
B = 128
RPT_ALL = 80
EP = RPT_ALL * B * NC * NS
NPT = NP // NS

_SC_MESH = dict(core_axis_name="c", subcore_axis_name="s",
                num_cores=NC, num_subcores=NS)


def _deg_body(dst3d, ones_hbm, z128, out_hbm, acc, idx_v, ones_v):
    c = lax.axis_index("c")
    s = lax.axis_index("s")
    w = c * NS + s
    pltpu.sync_copy(ones_hbm, ones_v)
    pltpu.sync_copy(dst3d.at[w], idx_v)
    pltpu.sync_copy(z128.at[pl.ds(s * NPT, NPT)], acc.at[pl.ds(s * NPT, NPT)])
    plsc.subcore_barrier()

    def body(j, carry):
        pltpu.sync_copy(ones_v, acc.at[idx_v.at[j]], add=True)
        return carry

    lax.fori_loop(0, RPT_ALL, body, 0)
    plsc.subcore_barrier()
    pltpu.sync_copy(acc.at[pl.ds(s * NPT, NPT)],
                    out_hbm.at[c].at[pl.ds(s * NPT, NPT)])


def _sc_deg(dst3d, ones128, z128):
    return pl.kernel(
        _deg_body,
        out_type=jax.ShapeDtypeStruct((NC, NP, 128), jnp.float32),
        mesh=plsc.VectorSubcoreMesh(**_SC_MESH),
        scratch_types=[
            pltpu.VMEM_SHARED((NP, 128), jnp.float32),
            pltpu.VMEM((RPT_ALL, B), jnp.int32),
            pltpu.VMEM((B, 128), jnp.float32),
        ],
    )(dst3d, ones128, z128)


NBUF = 2
HALVES = 2
RPH = RPT_ALL // HALVES


def _scatter_body(nch, W, table, src3d, dst3d, zW, out_hbm,
                  acc, src_v, dst_v, *rest):
    bufs = rest[:NBUF]
    gsem = rest[NBUF:2 * NBUF]
    c = lax.axis_index("c")
    s = lax.axis_index("s")
    w = c * NS + s
    for ch in range(nch):
        pltpu.sync_copy(zW.at[pl.ds(s * NPT, NPT)], acc.at[pl.ds(s * NPT, NPT)])
        plsc.subcore_barrier()

        for half in range(HALVES):
            pltpu.sync_copy(src3d.at[w].at[pl.ds(half * RPH, RPH)], src_v)
            pltpu.sync_copy(dst3d.at[w].at[pl.ds(half * RPH, RPH)], dst_v)
            for b in range(NBUF):
                pltpu.async_copy(table.at[ch].at[src_v.at[b]], bufs[b],
                                 gsem[b])

            def body(k, carry):
                for b in range(NBUF):
                    j = k * NBUF + b
                    pltpu.make_async_copy(table.at[ch].at[src_v.at[j]],
                                          bufs[b], gsem[b]).wait()
                    pltpu.sync_copy(bufs[b], acc.at[dst_v.at[j]], add=True)
                    nj = j + NBUF

                    @pl.when(nj < RPH)
                    def _():
                        pltpu.async_copy(table.at[ch].at[src_v.at[nj]],
                                         bufs[b], gsem[b])
                return carry

            lax.fori_loop(0, RPH // NBUF, body, 0)
        plsc.subcore_barrier()
        pltpu.sync_copy(acc.at[pl.ds(s * NPT, NPT)],
                        out_hbm.at[c].at[ch].at[pl.ds(s * NPT, NPT)])
        plsc.subcore_barrier()


def _sc_scatter(table, src3d, dst3d, zW):
    nch, _, W = table.shape
    return pl.kernel(
        functools.partial(_scatter_body, nch, W),
        out_type=jax.ShapeDtypeStruct((NC, nch, NP, W), jnp.float32),
        mesh=plsc.VectorSubcoreMesh(**_SC_MESH),
        scratch_types=(
            [pltpu.VMEM_SHARED((NP, W), jnp.float32),
             pltpu.VMEM((RPH, B), jnp.int32),
             pltpu.VMEM((RPH, B), jnp.int32)]
            + [pltpu.VMEM((B, W), jnp.float32)] * NBUF
            + [pltpu.SemaphoreType.DMA] * NBUF
        ),
    )(table, src3d, dst3d, zW)


GPT = G // (NC * NS)
WIN = 128


def _pool_body(h_flat, batch_hbm, z_acc, out_flat, b_v, win_v, acc_v):
    c = lax.axis_index("c")
    s = lax.axis_index("s")
    w = c * NS + s
    g0 = w * GPT
    pltpu.sync_copy(batch_hbm, b_v)
    pltpu.sync_copy(z_acc, acc_v)

    def count_lt(g):
        def cbody(i, cnt):
            v = b_v[pl.ds(i * 16, 16)]
            return cnt + jnp.where(v < g, 1.0, 0.0)
        return jnp.sum(lax.fori_loop(0, N // 16, cbody,
                                     jnp.zeros((16,), jnp.float32))
                       ).astype(jnp.int32)

    bounds = [count_lt(g0 + t) for t in range(GPT + 1)]
    for gg in range(GPT):
        start, end = bounds[gg], bounds[gg + 1]
        nwin = (end - start + WIN - 1) // WIN

        def wbody(t, carry, start=start, end=end, gg=gg):
            ptr = start + t * WIN
            wstart = jnp.minimum(ptr, N - WIN)
            pltpu.sync_copy(h_flat.at[pl.ds(wstart * 512, WIN * 512)], win_v)
            lo = ptr - wstart
            nn = jnp.minimum(end - ptr, WIN)

            def nbody(i, carry2):
                r = lo + i
                for k in range(32):
                    a = acc_v[pl.ds(gg * 512 + k * 16, 16)]
                    v = win_v[pl.ds(r * 512 + k * 16, 16)]
                    acc_v[pl.ds(gg * 512 + k * 16, 16)] = jnp.maximum(a, v)
                return carry2

            lax.fori_loop(0, nn, nbody, 0)
            return carry

        lax.fori_loop(0, nwin, wbody, 0)
    pltpu.sync_copy(acc_v, out_flat.at[pl.ds(g0 * 512, GPT * 512)])


def _sc_pool(h_flat, batch, z_acc):
    return pl.kernel(
        _pool_body,
        out_type=jax.ShapeDtypeStruct((G * 512,), jnp.float32),
        compiler_params=pltpu.CompilerParams(needs_layout_passes=False),
        mesh=plsc.VectorSubcoreMesh(**_SC_MESH),
        scratch_types=[
            pltpu.VMEM((N,), jnp.int32),
            pltpu.VMEM((WIN * 512,), jnp.float32),
            pltpu.VMEM((GPT * 512,), jnp.float32),
        ],
    )(h_flat, batch, z_acc)


R = 1024


def _dinv_body(degparts_ref, o_ref):
    deg = degparts_ref[0, :, 0:1] + degparts_ref[1, :, 0:1] + 1.0
    o_ref[...] = lax.rsqrt(deg)


def _tc_dinv(degparts):
    return pl.pallas_call(
        _dinv_body,
        grid=(NP // R,),
        in_specs=[pl.BlockSpec((NC, R, 128), lambda i: (0, i, 0))],
        out_specs=pl.BlockSpec((R, 1), lambda i: (i, 0)),
        out_shape=jax.ShapeDtypeStruct((NP, 1), jnp.float32),
    )(degparts)


def _layer1_body(x_ref, w_ref, dinv_ref, o_ref):
    xw = jnp.dot(x_ref[...], w_ref[0], preferred_element_type=jnp.float32)
    o_ref[0] = xw * dinv_ref[...]


def _tc_layer1(x, w, dinv, nch_out):
    dout = w.shape[1]
    Wc = dout // nch_out
    wch = w.reshape(128, nch_out, Wc).transpose(1, 0, 2)
    return pl.pallas_call(
        _layer1_body,
        grid=(NP // R, nch_out),
        in_specs=[
            pl.BlockSpec((R, 128), lambda i, j: (i, 0)),
            pl.BlockSpec((1, 128, Wc), lambda i, j: (j, 0, 0)),
            pl.BlockSpec((R, 1), lambda i, j: (i, 0)),
        ],
        out_specs=pl.BlockSpec((1, R, Wc), lambda i, j: (j, i, 0)),
        out_shape=jax.ShapeDtypeStruct((nch_out, NP, Wc), jnp.float32),
    )(x, wch, dinv)


def _layer_body(nch_in, Wc_in, acc_ref, xwp_ref, dinv_ref, b_ref, w_ref, o_ref):
    dinv = dinv_ref[...]
    parts = []
    for cc in range(nch_in):
        asum = acc_ref[0, cc] + acc_ref[1, cc]
        hp = dinv * (asum + xwp_ref[cc]) + b_ref[0, cc * Wc_in:(cc + 1) * Wc_in]
        parts.append(jnp.maximum(hp, 0.0))
    h = jnp.concatenate(parts, axis=1) if nch_in > 1 else parts[0]
    xw = jnp.dot(h, w_ref[...], preferred_element_type=jnp.float32)
    o_ref[0] = xw * dinv


def _tc_layer(acc, xwp, dinv, b, w, nch_out):
    _, nch_in, _, Wc_in = acc.shape
    din, dout = w.shape
    Wc = dout // nch_out
    return pl.pallas_call(
        functools.partial(_layer_body, nch_in, Wc_in),
        grid=(NP // R, nch_out),
        in_specs=[
            pl.BlockSpec((NC, nch_in, R, Wc_in), lambda i, j: (0, 0, i, 0)),
            pl.BlockSpec((nch_in, R, Wc_in), lambda i, j: (0, i, 0)),
            pl.BlockSpec((R, 1), lambda i, j: (i, 0)),
            pl.BlockSpec((1, din), lambda i, j: (0, 0)),
            pl.BlockSpec((din, Wc), lambda i, j: (0, j)),
        ],
        out_specs=pl.BlockSpec((1, R, Wc), lambda i, j: (j, i, 0)),
        out_shape=jax.ShapeDtypeStruct((nch_out, NP, Wc), jnp.float32),
    )(acc, xwp, dinv, b, w)


def _final_body(nch_in, Wc_in, acc_ref, xwp_ref, dinv_ref, b_ref, o_ref):
    dinv = dinv_ref[...]
    parts = []
    for cc in range(nch_in):
        asum = acc_ref[0, cc] + acc_ref[1, cc]
        hp = dinv * (asum + xwp_ref[cc]) + b_ref[0, cc * Wc_in:(cc + 1) * Wc_in]
        parts.append(jnp.maximum(hp, 0.0))
    o_ref[...] = jnp.concatenate(parts, axis=1)


def _tc_final(acc, xwp, dinv, b):
    _, nch_in, _, Wc_in = acc.shape
    din = nch_in * Wc_in
    return pl.pallas_call(
        functools.partial(_final_body, nch_in, Wc_in),
        grid=(NP // R,),
        in_specs=[
            pl.BlockSpec((NC, nch_in, R, Wc_in), lambda i: (0, 0, i, 0)),
            pl.BlockSpec((nch_in, R, Wc_in), lambda i: (0, i, 0)),
            pl.BlockSpec((R, 1), lambda i: (i, 0)),
            pl.BlockSpec((1, din), lambda i: (0, 0)),
        ],
        out_specs=pl.BlockSpec((R, din), lambda i: (i, 0)),
        out_shape=jax.ShapeDtypeStruct((NP, din), jnp.float32),
    )(acc, xwp, dinv, b)


GB = 16


def _pool3(y, L):
    L3 = (L - 3) // 3 + 1
    yr = y[:, :L3 * 3].reshape(y.shape[0], L3, 3, y.shape[2])
    return jnp.maximum(jnp.maximum(yr[:, :, 0], yr[:, :, 1]), yr[:, :, 2])


def _cell_body(x_ref, k1_ref, b1_ref, k2_ref, b2_ref, k3_ref, b3_ref, o_ref):
    x = x_ref[...]
    y1 = jnp.zeros((GB, 728, 32), jnp.float32)
    for j in range(8):
        y1 = y1 + x[:, j:j + 728, :] * k1_ref[j]
    y1 = jnp.maximum(y1 + b1_ref[...], 0.0)
    m1 = _pool3(y1, 728)

    x2 = jnp.concatenate([m1[:, j:j + 235, :] for j in range(8)], axis=2)
    z2 = jnp.dot(x2.reshape(GB * 235, 256), k2_ref[...],
                 preferred_element_type=jnp.float32).reshape(GB, 235, 64)
    y2 = jnp.maximum(z2 + b2_ref[...], 0.0)
    m2 = _pool3(y2, 235)

    x3 = jnp.concatenate([m2[:, j:j + 71, :] for j in range(8)], axis=2)
    z3 = jnp.dot(x3.reshape(GB * 71, 512), k3_ref[...],
                 preferred_element_type=jnp.float32).reshape(GB, 71, 128)
    y3 = jnp.maximum(z3 + b3_ref[...], 0.0)
    o_ref[...] = _pool3(y3, 71)


def _tc_cell(xg, k1, b1, k2, b2, k3, b3):
    return pl.pallas_call(
        _cell_body,
        grid=(G // GB,),
        in_specs=[
            pl.BlockSpec((GB, 735, 1), lambda i: (i, 0, 0)),
            pl.BlockSpec((8, 1, 32), lambda i: (0, 0, 0)),
            pl.BlockSpec((1, 1, 32), lambda i: (0, 0, 0)),
            pl.BlockSpec((256, 64), lambda i: (0, 0)),
            pl.BlockSpec((1, 1, 64), lambda i: (0, 0, 0)),
            pl.BlockSpec((512, 128), lambda i: (0, 0)),
            pl.BlockSpec((1, 1, 128), lambda i: (0, 0, 0)),
        ],
        out_specs=pl.BlockSpec((GB, 23, 128), lambda i: (i, 0, 0)),
        out_shape=jax.ShapeDtypeStruct((G, 23, 128), jnp.float32),
    )(xg, k1, b1, k2, b2, k3, b3)


def _head_body(p_ref, cf_ref, lw1, lb1, lw2, lb2, cfw, cfb,
               fw1a, fw1b, fb1, fw2, fb2, fw3, fb3, o_ref):
    mm = lambda a, b: jnp.dot(a, b[...], preferred_element_type=jnp.float32)
    d = jnp.maximum(mm(p_ref[...], lw1) + lb1[...], 0.0)
    d = jnp.maximum(mm(d, lw2) + lb2[...], 0.0)
    cv = mm(cf_ref[...], cfw) + cfb[...]
    f = jnp.maximum(mm(d, fw1a) + mm(cv, fw1b) + fb1[...], 0.0)
    f = jnp.maximum(mm(f, fw2) + fb2[...], 0.0)
    z = mm(f, fw3) + fb3[...]
    o_ref[...] = jax.nn.sigmoid(z)


def _tc_head(pooled, cflat, lW1, lB1, lW2, lB2, cFW2, cFB,
             fW1a, fW1b, fB1, fW2, fB2, fW3, fB3):
    args = (pooled, cflat, lW1, lB1[None], lW2, lB2[None], cFW2, cFB[None],
            fW1a, fW1b, fB1[None], fW2, fB2[None], fW3, fB3[None])
    return pl.pallas_call(
        _head_body,
        out_shape=jax.ShapeDtypeStruct((G, 1), jnp.float32),
    )(*args)


def kernel(x, edge_index, batch, cell, gW1, gB1, gW2, gB2, gW3, gB3,
           lW1, lB1, lW2, lB2, cK1, cb1, cK2, cb2, cK3, cb3, cFW, cFB,
           fW1, fB1, fW2, fB2, fW3, fB3):
    ei = edge_index.astype(jnp.int32)
    pad = N + jnp.arange(EP - E, dtype=jnp.int32) % (NP - N)
    ei = jnp.concatenate([ei, jnp.stack([pad, pad])], axis=1)
    src3d = ei[0].reshape(NC * NS, RPT_ALL, B)
    dst3d = ei[1].reshape(NC * NS, RPT_ALL, B)

    ones128 = jnp.ones((B, 128), jnp.float32)
    z128 = jnp.zeros((NP, 128), jnp.float32)
    zpool = jnp.zeros((GPT * 512,), jnp.float32)
    xp = jnp.zeros((NP, 128), jnp.float32).at[:N].set(x)

    degparts = _sc_deg(dst3d, ones128, z128)
    dinv = _tc_dinv(degparts)

    xwp1 = _tc_layer1(xp, gW1, dinv, 1)
    acc1 = _sc_scatter(xwp1, src3d, dst3d, z128)
    xwp2 = _tc_layer(acc1, xwp1, dinv, gB1[None], gW2, 2)
    acc2 = _sc_scatter(xwp2, src3d, dst3d, z128)
    xwp3 = _tc_layer(acc2, xwp2, dinv, gB2[None], gW3, 4)
    acc3 = _sc_scatter(xwp3, src3d, dst3d, z128)

    k1 = cK1.reshape(32, 8).T.reshape(8, 1, 32)
    k2 = cK2.transpose(2, 1, 0).reshape(256, 64)
    k3 = cK3.transpose(2, 1, 0).reshape(512, 128)
    cell_r, _ = lax.optimization_barrier((cell.reshape(G, 735, 1), xwp3))
    cfeat = _tc_cell(cell_r, k1, cb1.reshape(1, 1, 32),
                     k2, cb2.reshape(1, 1, 64), k3, cb3.reshape(1, 1, 128))
    cflat = cfeat.reshape(G, 23 * 128)
    cFW2 = cFW.reshape(128, 23, 128).transpose(1, 0, 2).reshape(23 * 128, 128)
    acc3, cflat = lax.optimization_barrier((acc3, cflat))

    h3 = _tc_final(acc3, xwp3, dinv, gB3[None])

    pooled = _sc_pool(h3.reshape(-1), batch.astype(jnp.int32), zpool)
    pooled = pooled.reshape(G, 512)

    return _tc_head(pooled, cflat, lW1, lB1, lW2, lB2, cFW2, cFB,
                    fW1[:128], fW1[128:], fB1, fW2, fB2, fW3, fB3)
